# baseline (speedup 1.0000x reference)
import functools

import jax
import jax.numpy as jnp
from jax import lax
from jax.experimental import pallas as pl
from jax.experimental.pallas import tpu as pltpu
from jax.experimental.pallas import tpu_sc as plsc

_N = 2500
_NPAD = 2560
_E = 160000
_NT = 16
_EPT = _E // _NT
_G = _EPT // 16
_STRIP = _NPAD // _NT
_SC = _STRIP // 16
_NCH = _NPAD // 16

_F32 = jnp.float32


def _z16():
    return jnp.zeros((16,), _F32)


def _newton_rsqrt(d):
    i = plsc.bitcast(d, jnp.int32)
    i = 0x5F3759DF - (i >> 1)
    y = plsc.bitcast(i, _F32)
    for _ in range(3):
        y = y * (1.5 - 0.5 * d * y * y)
    return y


def _elu(v):
    return jnp.where(v > 0, v, jnp.exp(v) - 1.0)


def _gcn_body(src_h, dst_h, xt_h, wb_h, wfc_h, ih_h, out_h,
              src_v, dst_v, wb_v, stab_v, out_v, deg_v, dinv_v,
              sstr_v, hstr_v, acc8_v, xk_v, wfcn_v, lg_v, lgall_v,
              idx8_v, sem0,
              sh_dacc, sh_s, sh_acc, sh_lg):
    t = lax.axis_index("s")
    base = t * _STRIP
    ones16 = jnp.full((16,), 1.0, _F32)

    pltpu.sync_copy(src_h.at[t], src_v)
    pltpu.sync_copy(dst_h.at[t], dst_v)
    pltpu.sync_copy(wb_h, wb_v)
    pltpu.sync_copy(ih_h.at[t], idx8_v)
    rf = [(jnp.int32(f) - t) & 7 for f in range(8)]
    _wchunks = [wb_v[pl.ds(i * 16, 16)] for i in range(7)]

    def _w(i):
        return _wchunks[i // 16][i % 16]

    def _zero_deg(i, c):
        deg_v[pl.ds(i * 16, 16)] = _z16()
        return c
    lax.fori_loop(0, _NCH, _zero_deg, 0)

    @plsc.parallel_loop(0, _G, unroll=5)
    def _deg(g):
        idx = dst_v[pl.ds(g * 16, 16)]
        plsc.addupdate_scatter(deg_v, [idx], ones16)

    pltpu.sync_copy(deg_v, sh_dacc.at[t])
    plsc.subcore_barrier()

    for c in range(_SC):
        dinv_v[pl.ds(c * 16, 16)] = _z16()

    def _red_deg(k, c):
        pltpu.sync_copy(sh_dacc.at[k, pl.ds(base, _STRIP)], xk_v.at[0])
        for cc in range(_SC):
            dinv_v[pl.ds(cc * 16, 16)] = (dinv_v[pl.ds(cc * 16, 16)]
                                          + xk_v[0, pl.ds(cc * 16, 16)])
        return c
    lax.fori_loop(0, _NT, _red_deg, 0)

    for c in range(_SC):
        d = dinv_v[pl.ds(c * 16, 16)] + 1.0
        dinv_v[pl.ds(c * 16, 16)] = _newton_rsqrt(d)

    for k in range(3):
        pltpu.sync_copy(xt_h.at[k, pl.ds(base, _STRIP)], xk_v.at[k])
    for c in range(_SC):
        dv = dinv_v[pl.ds(c * 16, 16)]
        xs = [xk_v[k, pl.ds(c * 16, 16)] for k in range(3)]
        for f in range(8):
            a = xs[0] * _w(0 * 8 + f)
            a = a + xs[1] * _w(1 * 8 + f)
            a = a + xs[2] * _w(2 * 8 + f)
            sstr_v[f, pl.ds(c * 16, 16)] = a * dv

    def _edge_pass():
        pltpu.sync_copy(sstr_v, sh_s.at[:, pl.ds(base, _STRIP)])
        for f in range(8):
            for c in range(_SC):
                hstr_v[f, pl.ds(c * 16, 16)] = _z16()
        pltpu.sync_copy(hstr_v, sh_acc.at[:, pl.ds(base, _STRIP)])
        plsc.subcore_barrier()
        cp = pltpu.async_copy(sh_s, stab_v, sem0)

        def _zero_out(i, c):
            for f in range(8):
                out_v[f, pl.ds(i * 16, 16)] = _z16()
            return c
        lax.fori_loop(0, _NCH, _zero_out, 0)
        cp.wait()

        @plsc.parallel_loop(0, _G, unroll=5)
        def _edges(g):
            s16 = src_v[pl.ds(g * 16, 16)]
            d16 = dst_v[pl.ds(g * 16, 16)]
            for f in range(8):
                v = plsc.load_gather(stab_v.at[f], [s16])
                plsc.addupdate_scatter(out_v.at[rf[f]], [d16], v)

        plsc.subcore_barrier()
        pltpu.sync_copy(out_v, sh_acc.at[idx8_v], add=True)
        plsc.subcore_barrier()
        pltpu.sync_copy(sh_acc.at[:, pl.ds(base, _STRIP)], acc8_v)

    _edge_pass()
    for f in range(8):
        b = _w(24 + f)
        for c in range(_SC):
            dv = dinv_v[pl.ds(c * 16, 16)]
            v = dv * (acc8_v[f, pl.ds(c * 16, 16)]
                      + sstr_v[f, pl.ds(c * 16, 16)]) + b
            hstr_v[f, pl.ds(c * 16, 16)] = _elu(v)

    for c in range(_SC):
        dv = dinv_v[pl.ds(c * 16, 16)]
        hk = [hstr_v[k, pl.ds(c * 16, 16)] for k in range(8)]
        for f in range(8):
            a = hk[0] * _w(32 + 0 * 8 + f)
            for k in range(1, 8):
                a = a + hk[k] * _w(32 + k * 8 + f)
            sstr_v[f, pl.ds(c * 16, 16)] = a * dv

    _edge_pass()
    for f in range(8):
        b = _w(96 + f)
        for c in range(_SC):
            dv = dinv_v[pl.ds(c * 16, 16)]
            v = dv * (acc8_v[f, pl.ds(c * 16, 16)]
                      + sstr_v[f, pl.ds(c * 16, 16)]) + b
            hstr_v[f, pl.ds(c * 16, 16)] = _elu(v)

    pltpu.sync_copy(wfc_h.at[pl.ds(t * _STRIP * 8, _STRIP * 8)], wfcn_v)
    iota = lax.iota(jnp.int32, 16)
    iota8 = iota * 8
    jvecs = [jnp.full((16,), j, jnp.int32) for j in range(4)]
    lg = _z16()
    accs = [_z16() for _ in range(4)]
    for f in range(8):
        for c in range(_SC):
            hv = hstr_v[f, pl.ds(c * 16, 16)]
            rowv = iota8 + (c * 128 + f)
            for j in range(4):
                wv = plsc.load_gather(wfcn_v, [rowv, jvecs[j]])
                accs[j] = accs[j] + hv * wv
    for j in range(4):
        sj = jnp.sum(accs[j])
        lg = jnp.where(iota == j, jnp.full((16,), sj, _F32), lg)
    lg_v[...] = lg
    pltpu.sync_copy(lg_v, sh_lg.at[t])
    plsc.subcore_barrier()

    @pl.when(t == 0)
    def _():
        pltpu.sync_copy(sh_lg, lgall_v)
        acc = lgall_v[0, ...]
        for k in range(1, _NT):
            acc = acc + lgall_v[k, ...]
        lg_v[...] = acc

        bfc = _z16()
        for j in range(4):
            bfc = jnp.where(iota == j, jnp.full((16,), _w(104 + j), _F32), bfc)
        x = lg_v[...] + bfc
        mask = iota < 4
        m = jnp.max(jnp.where(mask, x, jnp.full((16,), -3e38, _F32)))
        xm = x - m
        e = jnp.where(mask, jnp.exp(xm), _z16())
        s = jnp.sum(e)
        sv = jnp.full((16,), s, _F32)
        bits = plsc.bitcast(sv, jnp.int32)
        ex = ((bits >> 23) & 0xFF) - 127
        mant = plsc.bitcast((bits & 0x007FFFFF) | 0x3F800000, _F32)
        tq = (mant - 1.0) / (mant + 1.0)
        y = ex.astype(_F32) * 0.6931471805599453 + 2.0 * (tq + tq * tq * tq / 3.0)
        for _i in range(2):
            y = y + sv * jnp.exp(-y) - 1.0
        lg_v[...] = xm - y
        pltpu.sync_copy(lg_v, out_h)


_SCRATCH = [
    pltpu.VMEM((_EPT,), jnp.int32),
    pltpu.VMEM((_EPT,), jnp.int32),
    pltpu.VMEM((112,), _F32),
    pltpu.VMEM((8, _NPAD), _F32),
    pltpu.VMEM((8, _NPAD), _F32),
    pltpu.VMEM((_NPAD,), _F32),
    pltpu.VMEM((_STRIP,), _F32),
    pltpu.VMEM((8, _STRIP), _F32),
    pltpu.VMEM((8, _STRIP), _F32),
    pltpu.VMEM((8, _STRIP), _F32),
    pltpu.VMEM((3, _STRIP), _F32),
    pltpu.VMEM((_STRIP * 8, 4), _F32),
    pltpu.VMEM((16,), _F32),
    pltpu.VMEM((_NT, 16), _F32),
    pltpu.VMEM((8,), jnp.int32),
    pltpu.SemaphoreType.DMA,
    pltpu.VMEM_SHARED((_NT, _NPAD), _F32),
    pltpu.VMEM_SHARED((8, _NPAD), _F32),
    pltpu.VMEM_SHARED((8, _NPAD), _F32),
    pltpu.VMEM_SHARED((_NT, 16), _F32),
]

_gcn_sc = functools.partial(
    pl.kernel,
    out_type=jax.ShapeDtypeStruct((16,), _F32),
    mesh=plsc.VectorSubcoreMesh(core_axis_name="c", subcore_axis_name="s",
                                num_cores=1),
    scratch_types=_SCRATCH,
    compiler_params=pltpu.CompilerParams(needs_layout_passes=False,
                                         use_tc_tiling_on_sc=False),
)(_gcn_body)


def kernel(x, edge_index, W1, b1, W2, b2, Wfc, bfc):
    src = edge_index[0].reshape(_NT, _EPT)
    dst = edge_index[1].reshape(_NT, _EPT)
    xt = jnp.zeros((3, _NPAD), _F32).at[:, :_N].set(x.T)
    wb = jnp.concatenate([W1.reshape(-1), b1, W2.reshape(-1), b2, bfc,
                          jnp.zeros((4,), _F32)])
    wfc_p = jnp.zeros((_NPAD * 8, 4), _F32).at[:_N * 8].set(Wfc)
    ih = ((jnp.arange(16, dtype=jnp.int32)[:, None]
           + jnp.arange(8, dtype=jnp.int32)[None, :]) % 8)
    out16 = _gcn_sc(src, dst, xt, wb, wfc_p, ih)
    return out16[:4]


# speedup vs baseline: 1.0006x; 1.0006x over previous
import functools

import jax
import jax.numpy as jnp
from jax import lax
from jax.experimental import pallas as pl
from jax.experimental.pallas import tpu as pltpu
from jax.experimental.pallas import tpu_sc as plsc

_N = 2500
_NPAD = 2560
_E = 160000
_NT = 16
_EPT = _E // _NT
_G = _EPT // 16
_STRIP = _NPAD // _NT
_SC = _STRIP // 16
_NCH = _NPAD // 16

_F32 = jnp.float32


def _z16():
    return jnp.zeros((16,), _F32)


def _newton_rsqrt(d):
    i = plsc.bitcast(d, jnp.int32)
    i = 0x5F3759DF - (i >> 1)
    y = plsc.bitcast(i, _F32)
    for _ in range(3):
        y = y * (1.5 - 0.5 * d * y * y)
    return y


def _elu(v):
    return jnp.where(v > 0, v, jnp.exp(v) - 1.0)


def _gcn_body(src_h, dst_h, xt_h, wb_h, wfc_h, ih_h, out_h,
              src_v, dst_v, wb_v, stab_v, out_v, deg_v, dinv_v,
              sstr_v, hstr_v, acc8_v, xk_v, wfcn_v, lg_v, lgall_v,
              idx8_v, sem0,
              sh_dacc, sh_s, sh_acc, sh_lg):
    t = lax.axis_index("s")
    base = t * _STRIP
    ones16 = jnp.full((16,), 1.0, _F32)

    pltpu.sync_copy(src_h.at[t], src_v)
    pltpu.sync_copy(dst_h.at[t], dst_v)
    pltpu.sync_copy(wb_h, wb_v)
    pltpu.sync_copy(ih_h.at[t], idx8_v)
    _wchunks = [wb_v[pl.ds(i * 16, 16)] for i in range(7)]

    def _w(i):
        return _wchunks[i // 16][i % 16]

    def _zero_deg(i, c):
        deg_v[pl.ds(i * 16, 16)] = _z16()
        return c
    lax.fori_loop(0, _NCH, _zero_deg, 0)

    @plsc.parallel_loop(0, _G, unroll=5)
    def _deg(g):
        idx = dst_v[pl.ds(g * 16, 16)]
        plsc.addupdate_scatter(deg_v, [idx], ones16)

    pltpu.sync_copy(deg_v, sh_dacc.at[t])
    plsc.subcore_barrier()

    for c in range(_SC):
        dinv_v[pl.ds(c * 16, 16)] = _z16()

    def _red_deg(k, c):
        pltpu.sync_copy(sh_dacc.at[k, pl.ds(base, _STRIP)], xk_v.at[0])
        for cc in range(_SC):
            dinv_v[pl.ds(cc * 16, 16)] = (dinv_v[pl.ds(cc * 16, 16)]
                                          + xk_v[0, pl.ds(cc * 16, 16)])
        return c
    lax.fori_loop(0, _NT, _red_deg, 0)

    for c in range(_SC):
        d = dinv_v[pl.ds(c * 16, 16)] + 1.0
        dinv_v[pl.ds(c * 16, 16)] = _newton_rsqrt(d)

    for k in range(3):
        pltpu.sync_copy(xt_h.at[k, pl.ds(base, _STRIP)], xk_v.at[k])
    for c in range(_SC):
        dv = dinv_v[pl.ds(c * 16, 16)]
        xs = [xk_v[k, pl.ds(c * 16, 16)] for k in range(3)]
        for f in range(8):
            a = xs[0] * _w(0 * 8 + f)
            a = a + xs[1] * _w(1 * 8 + f)
            a = a + xs[2] * _w(2 * 8 + f)
            sstr_v[f, pl.ds(c * 16, 16)] = a * dv

    def _edge_pass():
        pltpu.sync_copy(sstr_v, sh_s.at[:, pl.ds(base, _STRIP)])
        for f in range(8):
            for c in range(_SC):
                hstr_v[f, pl.ds(c * 16, 16)] = _z16()
        pltpu.sync_copy(hstr_v, sh_acc.at[:, pl.ds(base, _STRIP)])
        plsc.subcore_barrier()
        cp = pltpu.async_copy(sh_s, stab_v, sem0)

        def _zero_out(i, c):
            for f in range(8):
                out_v[f, pl.ds(i * 16, 16)] = _z16()
            return c
        lax.fori_loop(0, _NCH, _zero_out, 0)
        cp.wait()

        @plsc.parallel_loop(0, _G, unroll=5)
        def _edges(g):
            s16 = src_v[pl.ds(g * 16, 16)]
            d16 = dst_v[pl.ds(g * 16, 16)]
            for f in range(8):
                v = plsc.load_gather(stab_v.at[f], [s16])
                plsc.addupdate_scatter(out_v.at[f], [d16], v)

        plsc.subcore_barrier()
        pltpu.sync_copy(out_v, sh_acc.at[idx8_v], add=True)
        plsc.subcore_barrier()
        pltpu.sync_copy(sh_acc.at[:, pl.ds(base, _STRIP)], acc8_v)

    _edge_pass()
    for f in range(8):
        b = _w(24 + f)
        for c in range(_SC):
            dv = dinv_v[pl.ds(c * 16, 16)]
            v = dv * (acc8_v[f, pl.ds(c * 16, 16)]
                      + sstr_v[f, pl.ds(c * 16, 16)]) + b
            hstr_v[f, pl.ds(c * 16, 16)] = _elu(v)

    for c in range(_SC):
        dv = dinv_v[pl.ds(c * 16, 16)]
        hk = [hstr_v[k, pl.ds(c * 16, 16)] for k in range(8)]
        for f in range(8):
            a = hk[0] * _w(32 + 0 * 8 + f)
            for k in range(1, 8):
                a = a + hk[k] * _w(32 + k * 8 + f)
            sstr_v[f, pl.ds(c * 16, 16)] = a * dv

    _edge_pass()
    for f in range(8):
        b = _w(96 + f)
        for c in range(_SC):
            dv = dinv_v[pl.ds(c * 16, 16)]
            v = dv * (acc8_v[f, pl.ds(c * 16, 16)]
                      + sstr_v[f, pl.ds(c * 16, 16)]) + b
            hstr_v[f, pl.ds(c * 16, 16)] = _elu(v)

    pltpu.sync_copy(wfc_h.at[pl.ds(t * _STRIP * 8, _STRIP * 8)], wfcn_v)
    iota = lax.iota(jnp.int32, 16)
    iota8 = iota * 8
    jvecs = [jnp.full((16,), j, jnp.int32) for j in range(4)]
    lg = _z16()
    accs = [_z16() for _ in range(4)]
    for f in range(8):
        for c in range(_SC):
            hv = hstr_v[f, pl.ds(c * 16, 16)]
            rowv = iota8 + (c * 128 + f)
            for j in range(4):
                wv = plsc.load_gather(wfcn_v, [rowv, jvecs[j]])
                accs[j] = accs[j] + hv * wv
    for j in range(4):
        sj = jnp.sum(accs[j])
        lg = jnp.where(iota == j, jnp.full((16,), sj, _F32), lg)
    lg_v[...] = lg
    pltpu.sync_copy(lg_v, sh_lg.at[t])
    plsc.subcore_barrier()

    @pl.when(t == 0)
    def _():
        pltpu.sync_copy(sh_lg, lgall_v)
        acc = lgall_v[0, ...]
        for k in range(1, _NT):
            acc = acc + lgall_v[k, ...]
        lg_v[...] = acc

        bfc = _z16()
        for j in range(4):
            bfc = jnp.where(iota == j, jnp.full((16,), _w(104 + j), _F32), bfc)
        x = lg_v[...] + bfc
        mask = iota < 4
        m = jnp.max(jnp.where(mask, x, jnp.full((16,), -3e38, _F32)))
        xm = x - m
        e = jnp.where(mask, jnp.exp(xm), _z16())
        s = jnp.sum(e)
        sv = jnp.full((16,), s, _F32)
        bits = plsc.bitcast(sv, jnp.int32)
        ex = ((bits >> 23) & 0xFF) - 127
        mant = plsc.bitcast((bits & 0x007FFFFF) | 0x3F800000, _F32)
        tq = (mant - 1.0) / (mant + 1.0)
        y = ex.astype(_F32) * 0.6931471805599453 + 2.0 * (tq + tq * tq * tq / 3.0)
        for _i in range(2):
            y = y + sv * jnp.exp(-y) - 1.0
        lg_v[...] = xm - y
        pltpu.sync_copy(lg_v, out_h)


_SCRATCH = [
    pltpu.VMEM((_EPT,), jnp.int32),
    pltpu.VMEM((_EPT,), jnp.int32),
    pltpu.VMEM((112,), _F32),
    pltpu.VMEM((8, _NPAD), _F32),
    pltpu.VMEM((8, _NPAD), _F32),
    pltpu.VMEM((_NPAD,), _F32),
    pltpu.VMEM((_STRIP,), _F32),
    pltpu.VMEM((8, _STRIP), _F32),
    pltpu.VMEM((8, _STRIP), _F32),
    pltpu.VMEM((8, _STRIP), _F32),
    pltpu.VMEM((3, _STRIP), _F32),
    pltpu.VMEM((_STRIP * 8, 4), _F32),
    pltpu.VMEM((16,), _F32),
    pltpu.VMEM((_NT, 16), _F32),
    pltpu.VMEM((8,), jnp.int32),
    pltpu.SemaphoreType.DMA,
    pltpu.VMEM_SHARED((_NT, _NPAD), _F32),
    pltpu.VMEM_SHARED((8, _NPAD), _F32),
    pltpu.VMEM_SHARED((8, _NPAD), _F32),
    pltpu.VMEM_SHARED((_NT, 16), _F32),
]

_gcn_sc = functools.partial(
    pl.kernel,
    out_type=jax.ShapeDtypeStruct((16,), _F32),
    mesh=plsc.VectorSubcoreMesh(core_axis_name="c", subcore_axis_name="s",
                                num_cores=1),
    scratch_types=_SCRATCH,
    compiler_params=pltpu.CompilerParams(needs_layout_passes=False,
                                         use_tc_tiling_on_sc=False),
)(_gcn_body)


def kernel(x, edge_index, W1, b1, W2, b2, Wfc, bfc):
    src = edge_index[0].reshape(_NT, _EPT)
    dst = edge_index[1].reshape(_NT, _EPT)
    xt = jnp.zeros((3, _NPAD), _F32).at[:, :_N].set(x.T)
    wb = jnp.concatenate([W1.reshape(-1), b1, W2.reshape(-1), b2, bfc,
                          jnp.zeros((4,), _F32)])
    wfc_p = jnp.zeros((_NPAD * 8, 4), _F32).at[:_N * 8].set(Wfc)
    ih = jnp.tile(jnp.arange(8, dtype=jnp.int32)[None, :], (16, 1))
    out16 = _gcn_sc(src, dst, xt, wb, wfc_p, ih)
    return out16[:4]


# speedup vs baseline: 1.0225x; 1.0219x over previous
import functools

import jax
import jax.numpy as jnp
from jax import lax
from jax.experimental import pallas as pl
from jax.experimental.pallas import tpu as pltpu
from jax.experimental.pallas import tpu_sc as plsc

_N = 2500
_NPAD = 2560
_E = 160000
_NT = 16
_EPT = _E // _NT
_G = _EPT // 16
_STRIP = _NPAD // _NT
_SC = _STRIP // 16
_NCH = _NPAD // 16

_F32 = jnp.float32


def _z16():
    return jnp.zeros((16,), _F32)


def _newton_rsqrt(d):
    i = plsc.bitcast(d, jnp.int32)
    i = 0x5F3759DF - (i >> 1)
    y = plsc.bitcast(i, _F32)
    for _ in range(3):
        y = y * (1.5 - 0.5 * d * y * y)
    return y


def _elu(v):
    return jnp.where(v > 0, v, jnp.exp(v) - 1.0)


def _gcn_body(src_h, dst_h, x_h, wb_h, wfc_h, ih_h, out_h,
              src_v, dst_v, wb_v, stab_v, out_v, deg_v, dinv_v,
              sstr_v, hstr_v, acc8_v, xk_v, xn_v, wfcn_v, lg_v, lgall_v,
              idx8_v, sem0,
              sh_dacc, sh_s, sh_acc, sh_lg):
    t = lax.axis_index("s")
    base = t * _STRIP
    ones16 = jnp.full((16,), 1.0, _F32)

    pltpu.sync_copy(src_h.at[t], src_v)
    pltpu.sync_copy(dst_h.at[t], dst_v)
    pltpu.sync_copy(wb_h, wb_v)
    pltpu.sync_copy(ih_h.at[t], idx8_v)
    _wchunks = [wb_v[pl.ds(i * 16, 16)] for i in range(7)]

    def _w(i):
        return _wchunks[i // 16][i % 16]

    iota = lax.iota(jnp.int32, 16)
    iota8 = iota * 8
    xstart = jnp.minimum(base, _N - _STRIP)
    xdelta = base - xstart
    masks = [(base + (c * 16) + iota) < _N for c in range(_SC)]

    def _zero_deg(i, c):
        deg_v[pl.ds(i * 16, 16)] = _z16()
        return c
    lax.fori_loop(0, _NCH, _zero_deg, 0)

    @plsc.parallel_loop(0, _G, unroll=5)
    def _deg(g):
        idx = dst_v[pl.ds(g * 16, 16)]
        plsc.addupdate_scatter(deg_v, [idx], ones16)

    pltpu.sync_copy(deg_v, sh_dacc.at[t])
    plsc.subcore_barrier()

    for c in range(_SC):
        dinv_v[pl.ds(c * 16, 16)] = _z16()

    def _red_deg(k, c):
        pltpu.sync_copy(sh_dacc.at[k, pl.ds(base, _STRIP)], xk_v.at[0])
        for cc in range(_SC):
            dinv_v[pl.ds(cc * 16, 16)] = (dinv_v[pl.ds(cc * 16, 16)]
                                          + xk_v[0, pl.ds(cc * 16, 16)])
        return c
    lax.fori_loop(0, _NT, _red_deg, 0)

    for c in range(_SC):
        d = dinv_v[pl.ds(c * 16, 16)] + 1.0
        dinv_v[pl.ds(c * 16, 16)] = _newton_rsqrt(d)

    pltpu.sync_copy(x_h.at[pl.ds(xstart, _STRIP)], xn_v)
    kvecs = [jnp.full((16,), k, jnp.int32) for k in range(3)]
    for c in range(_SC):
        dv = dinv_v[pl.ds(c * 16, 16)]
        rowx = jnp.minimum(iota + (c * 16) + xdelta, _STRIP - 1)
        xs = [plsc.load_gather(xn_v, [rowx, kvecs[k]]) for k in range(3)]
        for f in range(8):
            a = xs[0] * _w(0 * 8 + f)
            a = a + xs[1] * _w(1 * 8 + f)
            a = a + xs[2] * _w(2 * 8 + f)
            sstr_v[f, pl.ds(c * 16, 16)] = a * dv

    def _edge_pass():
        pltpu.sync_copy(sstr_v, sh_s.at[:, pl.ds(base, _STRIP)])
        for f in range(8):
            for c in range(_SC):
                hstr_v[f, pl.ds(c * 16, 16)] = _z16()
        pltpu.sync_copy(hstr_v, sh_acc.at[:, pl.ds(base, _STRIP)])
        plsc.subcore_barrier()
        cp = pltpu.async_copy(sh_s, stab_v, sem0)

        def _zero_out(i, c):
            for f in range(8):
                out_v[f, pl.ds(i * 16, 16)] = _z16()
            return c
        lax.fori_loop(0, _NCH, _zero_out, 0)
        cp.wait()

        @plsc.parallel_loop(0, _G, unroll=5)
        def _edges(g):
            s16 = src_v[pl.ds(g * 16, 16)]
            d16 = dst_v[pl.ds(g * 16, 16)]
            for f in range(8):
                v = plsc.load_gather(stab_v.at[f], [s16])
                plsc.addupdate_scatter(out_v.at[f], [d16], v)

        plsc.subcore_barrier()
        pltpu.sync_copy(out_v, sh_acc.at[idx8_v], add=True)
        plsc.subcore_barrier()
        pltpu.sync_copy(sh_acc.at[:, pl.ds(base, _STRIP)], acc8_v)

    _edge_pass()
    for f in range(8):
        b = _w(24 + f)
        for c in range(_SC):
            dv = dinv_v[pl.ds(c * 16, 16)]
            v = dv * (acc8_v[f, pl.ds(c * 16, 16)]
                      + sstr_v[f, pl.ds(c * 16, 16)]) + b
            hstr_v[f, pl.ds(c * 16, 16)] = _elu(v)

    for c in range(_SC):
        dv = dinv_v[pl.ds(c * 16, 16)]
        hk = [hstr_v[k, pl.ds(c * 16, 16)] for k in range(8)]
        for f in range(8):
            a = hk[0] * _w(32 + 0 * 8 + f)
            for k in range(1, 8):
                a = a + hk[k] * _w(32 + k * 8 + f)
            sstr_v[f, pl.ds(c * 16, 16)] = a * dv

    _edge_pass()
    for f in range(8):
        b = _w(96 + f)
        for c in range(_SC):
            dv = dinv_v[pl.ds(c * 16, 16)]
            v = dv * (acc8_v[f, pl.ds(c * 16, 16)]
                      + sstr_v[f, pl.ds(c * 16, 16)]) + b
            hstr_v[f, pl.ds(c * 16, 16)] = jnp.where(masks[c], _elu(v), 0.0)

    pltpu.sync_copy(wfc_h.at[pl.ds(xstart * 8, _STRIP * 8)], wfcn_v)
    jvecs = [jnp.full((16,), j, jnp.int32) for j in range(4)]
    lg = _z16()
    accs = [_z16() for _ in range(4)]
    wdelta = xdelta * 8
    for f in range(8):
        for c in range(_SC):
            hv = hstr_v[f, pl.ds(c * 16, 16)]
            rowv = jnp.minimum(iota8 + (c * 128 + f) + wdelta, _STRIP * 8 - 1)
            for j in range(4):
                wv = plsc.load_gather(wfcn_v, [rowv, jvecs[j]])
                accs[j] = accs[j] + hv * wv
    for j in range(4):
        sj = jnp.sum(accs[j])
        lg = jnp.where(iota == j, jnp.full((16,), sj, _F32), lg)
    lg_v[...] = lg
    pltpu.sync_copy(lg_v, sh_lg.at[t])
    plsc.subcore_barrier()

    @pl.when(t == 0)
    def _():
        pltpu.sync_copy(sh_lg, lgall_v)
        acc = lgall_v[0, ...]
        for k in range(1, _NT):
            acc = acc + lgall_v[k, ...]
        lg_v[...] = acc

        bfc = _z16()
        for j in range(4):
            bfc = jnp.where(iota == j, jnp.full((16,), _w(104 + j), _F32), bfc)
        x = lg_v[...] + bfc
        mask = iota < 4
        m = jnp.max(jnp.where(mask, x, jnp.full((16,), -3e38, _F32)))
        xm = x - m
        e = jnp.where(mask, jnp.exp(xm), _z16())
        s = jnp.sum(e)
        sv = jnp.full((16,), s, _F32)
        bits = plsc.bitcast(sv, jnp.int32)
        ex = ((bits >> 23) & 0xFF) - 127
        mant = plsc.bitcast((bits & 0x007FFFFF) | 0x3F800000, _F32)
        tq = (mant - 1.0) / (mant + 1.0)
        y = ex.astype(_F32) * 0.6931471805599453 + 2.0 * (tq + tq * tq * tq / 3.0)
        for _i in range(2):
            y = y + sv * jnp.exp(-y) - 1.0
        lg_v[...] = xm - y
        pltpu.sync_copy(lg_v, out_h)


_SCRATCH = [
    pltpu.VMEM((_EPT,), jnp.int32),
    pltpu.VMEM((_EPT,), jnp.int32),
    pltpu.VMEM((112,), _F32),
    pltpu.VMEM((8, _NPAD), _F32),
    pltpu.VMEM((8, _NPAD), _F32),
    pltpu.VMEM((_NPAD,), _F32),
    pltpu.VMEM((_STRIP,), _F32),
    pltpu.VMEM((8, _STRIP), _F32),
    pltpu.VMEM((8, _STRIP), _F32),
    pltpu.VMEM((8, _STRIP), _F32),
    pltpu.VMEM((3, _STRIP), _F32),
    pltpu.VMEM((_STRIP, 3), _F32),
    pltpu.VMEM((_STRIP * 8, 4), _F32),
    pltpu.VMEM((16,), _F32),
    pltpu.VMEM((_NT, 16), _F32),
    pltpu.VMEM((8,), jnp.int32),
    pltpu.SemaphoreType.DMA,
    pltpu.VMEM_SHARED((_NT, _NPAD), _F32),
    pltpu.VMEM_SHARED((8, _NPAD), _F32),
    pltpu.VMEM_SHARED((8, _NPAD), _F32),
    pltpu.VMEM_SHARED((_NT, 16), _F32),
]

_gcn_sc = functools.partial(
    pl.kernel,
    out_type=jax.ShapeDtypeStruct((16,), _F32),
    mesh=plsc.VectorSubcoreMesh(core_axis_name="c", subcore_axis_name="s",
                                num_cores=1),
    scratch_types=_SCRATCH,
    compiler_params=pltpu.CompilerParams(needs_layout_passes=False,
                                         use_tc_tiling_on_sc=False),
)(_gcn_body)


def kernel(x, edge_index, W1, b1, W2, b2, Wfc, bfc):
    src = edge_index[0].reshape(_NT, _EPT)
    dst = edge_index[1].reshape(_NT, _EPT)
    wb = jnp.concatenate([W1.reshape(-1), b1, W2.reshape(-1), b2, bfc,
                          jnp.zeros((4,), _F32)])
    ih = jnp.tile(jnp.arange(8, dtype=jnp.int32)[None, :], (16, 1))
    out16 = _gcn_sc(src, dst, x, wb, Wfc, ih)
    return out16[:4]


# speedup vs baseline: 1.2952x; 1.2667x over previous
import functools

import jax
import jax.numpy as jnp
from jax import lax
from jax.experimental import pallas as pl
from jax.experimental.pallas import tpu as pltpu
from jax.experimental.pallas import tpu_sc as plsc

_N = 2500
_NPAD = 2560
_E = 160000
_NT = 16
_EPT = _E // _NT
_G = _EPT // 16
_STRIP = _NPAD // _NT
_SC = _STRIP // 16
_NCH = _NPAD // 16

_F32 = jnp.float32


def _z16():
    return jnp.zeros((16,), _F32)


def _newton_rsqrt(d):
    i = plsc.bitcast(d, jnp.int32)
    i = 0x5F3759DF - (i >> 1)
    y = plsc.bitcast(i, _F32)
    for _ in range(3):
        y = y * (1.5 - 0.5 * d * y * y)
    return y


def _elu(v):
    return jnp.where(v > 0, v, jnp.exp(v) - 1.0)


def _gcn_body(src_h, dst_h, xt_h, wb_h, wfc_h, ih_h, out_h,
              src_v, dst_v, wb_v, stab_v, out_v, deg_v, dinv_v,
              sstr_v, hstr_v, acc8_v, xk_v, wfc_v, lg_v, lgall_v,
              idx8_v, sem0,
              sh_dacc, sh_s, sh_acc, sh_lg):
    t = lax.axis_index("s")
    base = t * _STRIP
    ones16 = jnp.full((16,), 1.0, _F32)

    pltpu.sync_copy(src_h.at[t], src_v)
    pltpu.sync_copy(dst_h.at[t], dst_v)
    pltpu.sync_copy(wb_h, wb_v)
    pltpu.sync_copy(ih_h.at[t], idx8_v)
    _wchunks = [wb_v[pl.ds(i * 16, 16)] for i in range(7)]

    def _w(i):
        return _wchunks[i // 16][i % 16]

    iota = lax.iota(jnp.int32, 16)

    def _zero_deg(i, c):
        deg_v[pl.ds(i * 16, 16)] = _z16()
        return c
    lax.fori_loop(0, _NCH, _zero_deg, 0)

    @plsc.parallel_loop(0, _G, unroll=5)
    def _deg(g):
        idx = dst_v[pl.ds(g * 16, 16)]
        plsc.addupdate_scatter(deg_v, [idx], ones16)

    pltpu.sync_copy(deg_v, sh_dacc.at[t])
    plsc.subcore_barrier()

    for c in range(_SC):
        dinv_v[pl.ds(c * 16, 16)] = _z16()

    def _red_deg(k, c):
        pltpu.sync_copy(sh_dacc.at[k, pl.ds(base, _STRIP)], xk_v.at[0])
        for cc in range(_SC):
            dinv_v[pl.ds(cc * 16, 16)] = (dinv_v[pl.ds(cc * 16, 16)]
                                          + xk_v[0, pl.ds(cc * 16, 16)])
        return c
    lax.fori_loop(0, _NT, _red_deg, 0)

    for c in range(_SC):
        d = dinv_v[pl.ds(c * 16, 16)] + 1.0
        dinv_v[pl.ds(c * 16, 16)] = _newton_rsqrt(d)

    for k in range(3):
        pltpu.sync_copy(xt_h.at[k, pl.ds(base, _STRIP)], xk_v.at[k])
    for c in range(_SC):
        dv = dinv_v[pl.ds(c * 16, 16)]
        xs = [xk_v[k, pl.ds(c * 16, 16)] for k in range(3)]
        for f in range(8):
            a = xs[0] * _w(0 * 8 + f)
            a = a + xs[1] * _w(1 * 8 + f)
            a = a + xs[2] * _w(2 * 8 + f)
            sstr_v[f, pl.ds(c * 16, 16)] = a * dv

    def _edge_pass():
        pltpu.sync_copy(sstr_v, sh_s.at[:, pl.ds(base, _STRIP)])
        for f in range(8):
            for c in range(_SC):
                hstr_v[f, pl.ds(c * 16, 16)] = _z16()
        pltpu.sync_copy(hstr_v, sh_acc.at[:, pl.ds(base, _STRIP)])
        plsc.subcore_barrier()
        cp = pltpu.async_copy(sh_s, stab_v, sem0)

        def _zero_out(i, c):
            for f in range(8):
                out_v[f, pl.ds(i * 16, 16)] = _z16()
            return c
        lax.fori_loop(0, _NCH, _zero_out, 0)
        cp.wait()

        @plsc.parallel_loop(0, _G, unroll=5)
        def _edges(g):
            s16 = src_v[pl.ds(g * 16, 16)]
            d16 = dst_v[pl.ds(g * 16, 16)]
            for f in range(8):
                v = plsc.load_gather(stab_v.at[f], [s16])
                plsc.addupdate_scatter(out_v.at[f], [d16], v)

        plsc.subcore_barrier()
        pltpu.sync_copy(out_v, sh_acc.at[idx8_v], add=True)
        plsc.subcore_barrier()
        pltpu.sync_copy(sh_acc.at[:, pl.ds(base, _STRIP)], acc8_v)

    _edge_pass()
    for f in range(8):
        b = _w(24 + f)
        for c in range(_SC):
            dv = dinv_v[pl.ds(c * 16, 16)]
            v = dv * (acc8_v[f, pl.ds(c * 16, 16)]
                      + sstr_v[f, pl.ds(c * 16, 16)]) + b
            hstr_v[f, pl.ds(c * 16, 16)] = _elu(v)

    for c in range(_SC):
        dv = dinv_v[pl.ds(c * 16, 16)]
        hk = [hstr_v[k, pl.ds(c * 16, 16)] for k in range(8)]
        for f in range(8):
            a = hk[0] * _w(32 + 0 * 8 + f)
            for k in range(1, 8):
                a = a + hk[k] * _w(32 + k * 8 + f)
            sstr_v[f, pl.ds(c * 16, 16)] = a * dv

    _edge_pass()
    for f in range(8):
        b = _w(96 + f)
        for c in range(_SC):
            dv = dinv_v[pl.ds(c * 16, 16)]
            v = dv * (acc8_v[f, pl.ds(c * 16, 16)]
                      + sstr_v[f, pl.ds(c * 16, 16)]) + b
            hstr_v[f, pl.ds(c * 16, 16)] = _elu(v)

    pltpu.sync_copy(wfc_h.at[:, pl.ds(base, _STRIP)], wfc_v)
    lg = _z16()
    for j in range(4):
        a = _z16()
        for f in range(8):
            for c in range(_SC):
                a = a + (hstr_v[f, pl.ds(c * 16, 16)]
                         * wfc_v[f * 4 + j, pl.ds(c * 16, 16)])
        sj = jnp.sum(a)
        lg = jnp.where(iota == j, jnp.full((16,), sj, _F32), lg)
    lg_v[...] = lg
    pltpu.sync_copy(lg_v, sh_lg.at[t])
    plsc.subcore_barrier()

    @pl.when(t == 0)
    def _():
        pltpu.sync_copy(sh_lg, lgall_v)
        acc = lgall_v[0, ...]
        for k in range(1, _NT):
            acc = acc + lgall_v[k, ...]
        lg_v[...] = acc

        bfc = _z16()
        for j in range(4):
            bfc = jnp.where(iota == j, jnp.full((16,), _w(104 + j), _F32), bfc)
        x = lg_v[...] + bfc
        mask = iota < 4
        m = jnp.max(jnp.where(mask, x, jnp.full((16,), -3e38, _F32)))
        xm = x - m
        e = jnp.where(mask, jnp.exp(xm), _z16())
        s = jnp.sum(e)
        sv = jnp.full((16,), s, _F32)
        bits = plsc.bitcast(sv, jnp.int32)
        ex = ((bits >> 23) & 0xFF) - 127
        mant = plsc.bitcast((bits & 0x007FFFFF) | 0x3F800000, _F32)
        tq = (mant - 1.0) / (mant + 1.0)
        y = ex.astype(_F32) * 0.6931471805599453 + 2.0 * (tq + tq * tq * tq / 3.0)
        for _i in range(2):
            y = y + sv * jnp.exp(-y) - 1.0
        lg_v[...] = xm - y
        pltpu.sync_copy(lg_v, out_h)


_SCRATCH = [
    pltpu.VMEM((_EPT,), jnp.int32),
    pltpu.VMEM((_EPT,), jnp.int32),
    pltpu.VMEM((112,), _F32),
    pltpu.VMEM((8, _NPAD), _F32),
    pltpu.VMEM((8, _NPAD), _F32),
    pltpu.VMEM((_NPAD,), _F32),
    pltpu.VMEM((_STRIP,), _F32),
    pltpu.VMEM((8, _STRIP), _F32),
    pltpu.VMEM((8, _STRIP), _F32),
    pltpu.VMEM((8, _STRIP), _F32),
    pltpu.VMEM((3, _STRIP), _F32),
    pltpu.VMEM((32, _STRIP), _F32),
    pltpu.VMEM((16,), _F32),
    pltpu.VMEM((_NT, 16), _F32),
    pltpu.VMEM((8,), jnp.int32),
    pltpu.SemaphoreType.DMA,
    pltpu.VMEM_SHARED((_NT, _NPAD), _F32),
    pltpu.VMEM_SHARED((8, _NPAD), _F32),
    pltpu.VMEM_SHARED((8, _NPAD), _F32),
    pltpu.VMEM_SHARED((_NT, 16), _F32),
]

_gcn_sc = functools.partial(
    pl.kernel,
    out_type=jax.ShapeDtypeStruct((16,), _F32),
    mesh=plsc.VectorSubcoreMesh(core_axis_name="c", subcore_axis_name="s",
                                num_cores=1),
    scratch_types=_SCRATCH,
    compiler_params=pltpu.CompilerParams(needs_layout_passes=False,
                                         use_tc_tiling_on_sc=False),
)(_gcn_body)


def kernel(x, edge_index, W1, b1, W2, b2, Wfc, bfc):
    src = edge_index[0].reshape(_NT, _EPT)
    dst = edge_index[1].reshape(_NT, _EPT)
    xt = jnp.zeros((3, _NPAD), _F32).at[:, :_N].set(x.T)
    wb = jnp.concatenate([W1.reshape(-1), b1, W2.reshape(-1), b2, bfc,
                          jnp.zeros((4,), _F32)])
    wfc_r = jnp.zeros((32, _NPAD), _F32).at[:, :_N].set(
        Wfc.reshape(_N, 8, 4).transpose(1, 2, 0).reshape(32, _N))
    ih = jnp.tile(jnp.arange(8, dtype=jnp.int32)[None, :], (16, 1))
    out16 = _gcn_sc(src, dst, xt, wb, wfc_r, ih)
    return out16[:4]
